# baseline (device time: 38747 ns/iter reference)
import jax
import jax.numpy as jnp
from jax import lax
from jax.experimental import pallas as pl
from jax.experimental.pallas import tpu as pltpu

N_DEV = 4
SQ = 256
D = 1024
HQ = 8
DH = 128
HALF = D // 2
SCALE = 0.08838834764831843


def kernel(x, Wq, Wo, K_ext, V_ext):
    x2 = x.reshape(SQ, D)
    k3 = K_ext.reshape(-1, HQ, DH)
    v3 = V_ext.reshape(-1, HQ, DH)
    skv = k3.shape[0]

    def body(
        x_ref,
        wq_ref,
        wo_ref,
        k_hbm,
        v_hbm,
        out_ref,
        kbuf,
        vbuf,
        qb,
        send_o,
        send_l,
        recv_o,
        recv_l,
        kv_sems,
        send_sems_o,
        send_sems_l,
        recv_sems_o,
        recv_sems_l,
    ):
        my_i = lax.axis_index("i")

        NCHUNK = 4
        CS = skv // NCHUNK

        def kv_copies(h):
            slot = h % 2
            copies = []
            for c in range(NCHUNK):
                rows = pl.ds(c * CS, CS)
                copies.append(pltpu.make_async_copy(
                    k_hbm.at[rows, h, :],
                    kbuf.at[slot, rows, :],
                    kv_sems.at[slot, 0, c],
                ))
                copies.append(pltpu.make_async_copy(
                    v_hbm.at[rows, h, :],
                    vbuf.at[slot, rows, :],
                    kv_sems.at[slot, 1, c],
                ))
            return copies

        def peer_copies(half, with_l):
            copies = []
            for d in (1, 2, 3):
                peer = (my_i + d) % N_DEV
                copies.append(pltpu.make_async_remote_copy(
                    src_ref=send_o.at[half],
                    dst_ref=recv_o.at[d - 1, half],
                    send_sem=send_sems_o.at[d - 1, half],
                    recv_sem=recv_sems_o.at[d - 1, half],
                    device_id=(peer,),
                    device_id_type=pl.DeviceIdType.MESH,
                ))
                if with_l:
                    copies.append(pltpu.make_async_remote_copy(
                        src_ref=send_l,
                        dst_ref=recv_l.at[d - 1],
                        send_sem=send_sems_l.at[d - 1],
                        recv_sem=recv_sems_l.at[d - 1],
                        device_id=(peer,),
                        device_id_type=pl.DeviceIdType.MESH,
                    ))
            return copies

        barrier = pltpu.get_barrier_semaphore()
        for d in (1, 2, 3):
            pl.semaphore_signal(
                barrier,
                inc=1,
                device_id=((my_i + d) % N_DEV,),
                device_id_type=pl.DeviceIdType.MESH,
            )
        pl.semaphore_wait(barrier, 3)

        for c in kv_copies(0):
            c.start()
        q = jnp.dot(
            x_ref[...].astype(jnp.bfloat16),
            wq_ref[...].astype(jnp.bfloat16),
            preferred_element_type=jnp.float32,
        )
        qb[...] = (q * SCALE).astype(jnp.bfloat16)

        for h in range(HQ):
            if h + 1 < HQ:
                for c in kv_copies(h + 1):
                    c.start()
            for c in kv_copies(h):
                c.wait()
            slot = h % 2
            half, hh = divmod(h, HQ // 2)
            kb = kbuf[slot].astype(jnp.bfloat16)
            vb = vbuf[slot].astype(jnp.bfloat16)
            s = lax.dot_general(
                qb[:, h * DH:(h + 1) * DH],
                kb,
                (((1,), (1,)), ((), ())),
                preferred_element_type=jnp.float32,
            )
            p = jnp.exp(s)
            send_l[:, h:h + 1] = jnp.sum(p, axis=1, keepdims=True)
            o = jnp.dot(
                p.astype(jnp.bfloat16), vb, preferred_element_type=jnp.float32
            )
            send_o[half, :, pl.ds(hh * DH, DH)] = o.astype(jnp.bfloat16)
            if h == HQ // 2 - 1:
                for c in peer_copies(0, with_l=False):
                    c.start()

        for c in peer_copies(1, with_l=True):
            c.start()
        for c in peer_copies(0, with_l=False) + peer_copies(1, with_l=True):
            c.wait()

        l_tot = send_l[...] + recv_l[0] + recv_l[1] + recv_l[2]
        cols_out = []
        for hf in range(2):
            o_tot = (
                send_o[hf].astype(jnp.float32)
                + recv_o[0, hf].astype(jnp.float32)
                + recv_o[1, hf].astype(jnp.float32)
                + recv_o[2, hf].astype(jnp.float32)
            )
            for hq in range(HQ // 2):
                hg = hf * (HQ // 2) + hq
                cols_out.append(
                    (
                        o_tot[:, hq * DH:(hq + 1) * DH] / l_tot[:, hg:hg + 1]
                    ).astype(jnp.bfloat16)
                )
        attn = jnp.concatenate(cols_out, axis=1)
        out_ref[...] = jnp.dot(
            attn,
            wo_ref[...].astype(jnp.bfloat16),
            preferred_element_type=jnp.float32,
        )

    out = pl.pallas_call(
        body,
        out_shape=jax.ShapeDtypeStruct((SQ, D), jnp.float32),
        in_specs=[
            pl.BlockSpec(memory_space=pltpu.VMEM),
            pl.BlockSpec(memory_space=pltpu.VMEM),
            pl.BlockSpec(memory_space=pltpu.VMEM),
            pl.BlockSpec(memory_space=pl.MemorySpace.ANY),
            pl.BlockSpec(memory_space=pl.MemorySpace.ANY),
        ],
        out_specs=pl.BlockSpec(memory_space=pltpu.VMEM),
        scratch_shapes=[
            pltpu.VMEM((2, skv, DH), jnp.float32),
            pltpu.VMEM((2, skv, DH), jnp.float32),
            pltpu.VMEM((SQ, D), jnp.bfloat16),
            pltpu.VMEM((2, SQ, HALF), jnp.bfloat16),
            pltpu.VMEM((SQ, HQ), jnp.float32),
            pltpu.VMEM((N_DEV - 1, 2, SQ, HALF), jnp.bfloat16),
            pltpu.VMEM((N_DEV - 1, SQ, HQ), jnp.float32),
            pltpu.SemaphoreType.DMA((2, 2, 4)),
            pltpu.SemaphoreType.DMA((N_DEV - 1, 2)),
            pltpu.SemaphoreType.DMA((N_DEV - 1,)),
            pltpu.SemaphoreType.DMA((N_DEV - 1, 2)),
            pltpu.SemaphoreType.DMA((N_DEV - 1,)),
        ],
        compiler_params=pltpu.CompilerParams(
            collective_id=0,
            vmem_limit_bytes=100 * 1024 * 1024,
        ),
    )(x2, Wq, Wo, k3, v3)
    return out.reshape(1, SQ, D)


# device time: 34034 ns/iter; 1.1385x vs baseline; 1.1385x over previous
import jax
import jax.numpy as jnp
from jax import lax
from jax.experimental import pallas as pl
from jax.experimental.pallas import tpu as pltpu

N_DEV = 4
SQ = 256
D = 1024
HQ = 8
DH = 128
HALF = D // 2
SCALE = 0.08838834764831843
NSLOT = 3


def kernel(x, Wq, Wo, K_ext, V_ext):
    x2 = x.reshape(SQ, D)
    k3 = K_ext.reshape(-1, HQ, DH)
    v3 = V_ext.reshape(-1, HQ, DH)
    skv = k3.shape[0]

    def body(
        x_ref,
        wq_ref,
        wo_ref,
        k_hbm,
        v_hbm,
        out_ref,
        kbuf,
        vbuf,
        qb,
        send_o,
        send_l,
        recv_o,
        recv_l,
        kv_sems,
        send_sems_o,
        send_sems_l,
        recv_sems_o,
        recv_sems_l,
    ):
        my_i = lax.axis_index("i")

        def kv_copies(h):
            slot = h % NSLOT
            return (
                pltpu.make_async_copy(
                    k_hbm.at[:, h, :], kbuf.at[slot], kv_sems.at[slot, 0]
                ),
                pltpu.make_async_copy(
                    v_hbm.at[:, h, :], vbuf.at[slot], kv_sems.at[slot, 1]
                ),
            )

        def o_copies(h):
            copies = []
            for d in (1, 2, 3):
                copies.append(pltpu.make_async_remote_copy(
                    src_ref=send_o.at[h],
                    dst_ref=recv_o.at[d - 1, h],
                    send_sem=send_sems_o.at[d - 1, h],
                    recv_sem=recv_sems_o.at[d - 1, h],
                    device_id=((my_i + d) % N_DEV,),
                    device_id_type=pl.DeviceIdType.MESH,
                ))
            return copies

        def l_copies(half):
            copies = []
            for d in (1, 2, 3):
                copies.append(pltpu.make_async_remote_copy(
                    src_ref=send_l.at[half],
                    dst_ref=recv_l.at[d - 1, half],
                    send_sem=send_sems_l.at[d - 1, half],
                    recv_sem=recv_sems_l.at[d - 1, half],
                    device_id=((my_i + d) % N_DEV,),
                    device_id_type=pl.DeviceIdType.MESH,
                ))
            return copies

        barrier = pltpu.get_barrier_semaphore()
        for d in (1, 2, 3):
            pl.semaphore_signal(
                barrier,
                inc=1,
                device_id=((my_i + d) % N_DEV,),
                device_id_type=pl.DeviceIdType.MESH,
            )

        for hp in range(NSLOT - 1):
            for c in kv_copies(hp):
                c.start()
        q = jnp.dot(
            x_ref[...].astype(jnp.bfloat16),
            wq_ref[...].astype(jnp.bfloat16),
            preferred_element_type=jnp.float32,
        )
        qb[...] = (q * SCALE).astype(jnp.bfloat16)

        for h in range(HQ):
            if h + NSLOT - 1 < HQ:
                for c in kv_copies(h + NSLOT - 1):
                    c.start()
            for c in kv_copies(h):
                c.wait()
            slot = h % NSLOT
            kb = kbuf[slot].astype(jnp.bfloat16)
            vb = vbuf[slot].astype(jnp.bfloat16)
            s = lax.dot_general(
                qb[:, h * DH:(h + 1) * DH],
                kb,
                (((1,), (1,)), ((), ())),
                preferred_element_type=jnp.float32,
            )
            p = jnp.exp(s)
            half, hh = divmod(h, HQ // 2)
            send_l[half, :, hh:hh + 1] = jnp.sum(p, axis=1, keepdims=True)
            o = jnp.dot(
                p.astype(jnp.bfloat16), vb, preferred_element_type=jnp.float32
            )
            send_o[h] = o.astype(jnp.bfloat16)
            if h == 0:
                pl.semaphore_wait(barrier, 3)
            for c in o_copies(h):
                c.start()
            if h == HQ // 2 - 1:
                for c in l_copies(0):
                    c.start()

        for c in l_copies(1):
            c.start()

        def combine_half(half):
            for hh in range(HQ // 2):
                for c in o_copies(half * (HQ // 2) + hh):
                    c.wait_recv()
            for c in l_copies(half):
                c.wait_recv()
            l_tot = (
                send_l[half] + recv_l[0, half] + recv_l[1, half]
                + recv_l[2, half]
            )
            cols = []
            for hh in range(HQ // 2):
                h = half * (HQ // 2) + hh
                o_tot = (
                    send_o[h].astype(jnp.float32)
                    + recv_o[0, h].astype(jnp.float32)
                    + recv_o[1, h].astype(jnp.float32)
                    + recv_o[2, h].astype(jnp.float32)
                )
                cols.append((o_tot / l_tot[:, hh:hh + 1]).astype(jnp.bfloat16))
            attn = jnp.concatenate(cols, axis=1)
            return jnp.dot(
                attn,
                wo_ref[pl.ds(half * HALF, HALF), :].astype(jnp.bfloat16),
                preferred_element_type=jnp.float32,
            )

        acc = combine_half(0)
        out_ref[...] = acc + combine_half(1)

        for h in range(HQ):
            for c in o_copies(h):
                c.wait_send()
        for half in range(2):
            for c in l_copies(half):
                c.wait_send()

    out = pl.pallas_call(
        body,
        out_shape=jax.ShapeDtypeStruct((SQ, D), jnp.float32),
        in_specs=[
            pl.BlockSpec(memory_space=pltpu.VMEM),
            pl.BlockSpec(memory_space=pltpu.VMEM),
            pl.BlockSpec(memory_space=pltpu.VMEM),
            pl.BlockSpec(memory_space=pl.MemorySpace.ANY),
            pl.BlockSpec(memory_space=pl.MemorySpace.ANY),
        ],
        out_specs=pl.BlockSpec(memory_space=pltpu.VMEM),
        scratch_shapes=[
            pltpu.VMEM((NSLOT, skv, DH), jnp.float32),
            pltpu.VMEM((NSLOT, skv, DH), jnp.float32),
            pltpu.VMEM((SQ, D), jnp.bfloat16),
            pltpu.VMEM((HQ, SQ, DH), jnp.bfloat16),
            pltpu.VMEM((2, SQ, HQ // 2), jnp.float32),
            pltpu.VMEM((N_DEV - 1, HQ, SQ, DH), jnp.bfloat16),
            pltpu.VMEM((N_DEV - 1, 2, SQ, HQ // 2), jnp.float32),
            pltpu.SemaphoreType.DMA((NSLOT, 2)),
            pltpu.SemaphoreType.DMA((N_DEV - 1, HQ)),
            pltpu.SemaphoreType.DMA((N_DEV - 1, 2)),
            pltpu.SemaphoreType.DMA((N_DEV - 1, HQ)),
            pltpu.SemaphoreType.DMA((N_DEV - 1, 2)),
        ],
        compiler_params=pltpu.CompilerParams(
            collective_id=0,
            vmem_limit_bytes=100 * 1024 * 1024,
        ),
    )(x2, Wq, Wo, k3, v3)
    return out.reshape(1, SQ, D)
